# baseline (speedup 1.0000x reference)
import functools

import jax
import jax.numpy as jnp
from jax import lax
from jax.experimental import pallas as pl
from jax.experimental.pallas import tpu as pltpu
from jax.experimental.pallas import tpu_sc as plsc

B = 16384
D = 32
R = 8
V = 1_000_000
NSLAB = V // R
NC = 2
NS = 16
NW = NC * NS
B_PER_W = B // NW
G = 16
NGRP = B_PER_W // G

_mesh = plsc.VectorSubcoreMesh(core_axis_name="c", subcore_axis_name="s")


@functools.partial(
    pl.kernel,
    mesh=_mesh,
    out_type=jax.ShapeDtypeStruct((B, D), jnp.float32),
    compiler_params=pltpu.CompilerParams(needs_layout_passes=False),
    scratch_types=[
        pltpu.VMEM((B_PER_W,), jnp.int32),
        pltpu.VMEM((B_PER_W, D), jnp.float32),
        pltpu.SemaphoreType.DMA,
    ],
)
def _embed_gather(labels_hbm, table3_hbm, out_hbm, lab_v, out_v, sem):
    wid = lax.axis_index("s") * NC + lax.axis_index("c")
    base = wid * B_PER_W

    pltpu.sync_copy(labels_hbm.at[pl.ds(base, B_PER_W)], lab_v)

    @plsc.parallel_loop(0, NGRP, step=1, unroll=4)
    def _(g):
        lab = lab_v[pl.ds(g * G, G)]
        for l in range(G):
            lab_i = lab[l]
            t = lax.shift_right_logical(lab_i, 3)
            j = jnp.bitwise_and(lab_i, 7)
            pltpu.async_copy(table3_hbm.at[t, j], out_v.at[g * G + l], sem)

    pltpu.make_async_copy(out_hbm.at[pl.ds(base, B_PER_W)], out_v, sem).wait()
    pltpu.sync_copy(out_v, out_hbm.at[pl.ds(base, B_PER_W)])


def kernel(labels, train, table):
    del train
    table3 = table.reshape(NSLAB, R, D)
    return _embed_gather(labels.astype(jnp.int32), table3)


# speedup vs baseline: 1.0011x; 1.0011x over previous
import functools

import jax
import jax.numpy as jnp
from jax import lax
from jax.experimental import pallas as pl
from jax.experimental.pallas import tpu as pltpu
from jax.experimental.pallas import tpu_sc as plsc

B = 16384
D = 32
R = 8
V = 1_000_000
NSLAB = V // R
NC = 2
NS = 16
NW = NC * NS
B_PER_W = B // NW
G = 16
NGRP = B_PER_W // G

_mesh = plsc.VectorSubcoreMesh(core_axis_name="c", subcore_axis_name="s")


@functools.partial(
    pl.kernel,
    mesh=_mesh,
    out_type=jax.ShapeDtypeStruct((B, D), jnp.float32),
    compiler_params=pltpu.CompilerParams(needs_layout_passes=False),
    scratch_types=[
        pltpu.VMEM((B_PER_W,), jnp.int32),
        pltpu.VMEM((B_PER_W, D), jnp.float32),
        pltpu.SemaphoreType.DMA,
    ],
)
def _embed_gather(labels_hbm, table3_hbm, out_hbm, lab_v, out_v, sem):
    wid = lax.axis_index("s") * NC + lax.axis_index("c")
    base = wid * B_PER_W

    pltpu.sync_copy(labels_hbm.at[pl.ds(base, B_PER_W)], lab_v)

    def body(g, _):
        lab = lab_v[pl.ds(g * G, G)]
        for l in range(G):
            lab_i = lab[l]
            t = lax.shift_right_logical(lab_i, 3)
            j = jnp.bitwise_and(lab_i, 7)
            pltpu.async_copy(table3_hbm.at[t, j], out_v.at[g * G + l], sem)
        return 0

    lax.fori_loop(0, NGRP, body, 0)

    pltpu.make_async_copy(out_hbm.at[pl.ds(base, B_PER_W)], out_v, sem).wait()
    pltpu.sync_copy(out_v, out_hbm.at[pl.ds(base, B_PER_W)])


def kernel(labels, train, table):
    del train
    table3 = table.reshape(NSLAB, R, D)
    return _embed_gather(labels.astype(jnp.int32), table3)
